# baseline (speedup 1.0000x reference)
import functools

import jax
import jax.numpy as jnp
from jax import lax
from jax.experimental import pallas as pl
from jax.experimental.pallas import tpu as pltpu
from jax.experimental.pallas import tpu_sc as plsc

_LANE = 128
_NBUF = 2
_NW = 32


def _table_repack_body(t_ref, out_ref):
    x = t_ref[...]
    nb = t_ref.shape[1] // (4 * _LANE)
    for p in range(nb):
        s = jnp.concatenate(
            [
                x[:, 4 * _LANE * p + _LANE * j : 4 * _LANE * p + _LANE * (j + 1)]
                for j in range(4)
            ],
            axis=0,
        )
        out_ref[pl.ds(p * _LANE, _LANE), :] = s.T


def _table_repack(table_t, v, d):
    vb = 16384
    n_steps = -(-v // vb)
    v_pad = n_steps * vb
    return pl.pallas_call(
        _table_repack_body,
        grid=(n_steps,),
        in_specs=[pl.BlockSpec((d, vb), lambda i: (0, i))],
        out_specs=pl.BlockSpec((vb * d // _LANE, _LANE), lambda i: (i, 0)),
        out_shape=jax.ShapeDtypeStruct((v_pad * d // _LANE, _LANE), jnp.float32),
    )(table_t)


def _out_repack_body(in_ref, out_ref):
    bw = out_ref.shape[1]
    qn = out_ref.shape[0] // _LANE
    x3 = in_ref[...].reshape(bw, qn, _LANE)
    x3 = jnp.transpose(x3, (1, 0, 2))
    x3 = jnp.transpose(x3, (0, 2, 1))
    out_ref[...] = x3.reshape(out_ref.shape)


def _out_repack_alias_body(in_ref, y_ref, out_ref):
    del y_ref
    _out_repack_body(in_ref, out_ref)


def _out_repack_half(out2d_h, y_prev, b_total, ld, half):
    bw = 256
    b_half = b_total // 2
    n_steps = b_half // bw
    rows_per_blk = bw * ld // _LANE
    in_spec = pl.BlockSpec((rows_per_blk, _LANE), lambda i: (i, 0))
    if half == 0:
        return pl.pallas_call(
            _out_repack_body,
            grid=(n_steps,),
            in_specs=[in_spec],
            out_specs=pl.BlockSpec((ld, bw), lambda i: (0, i)),
            out_shape=jax.ShapeDtypeStruct((ld, b_total), jnp.float32),
        )(out2d_h)
    return pl.pallas_call(
        _out_repack_alias_body,
        grid=(n_steps,),
        in_specs=[in_spec, pl.BlockSpec(memory_space=pl.ANY)],
        out_specs=pl.BlockSpec((ld, bw), lambda i, off=n_steps: (0, i + off)),
        out_shape=jax.ShapeDtypeStruct((ld, b_total), jnp.float32),
        input_output_aliases={1: 0},
    )(out2d_h, y_prev)


def _sc_gather(idx2d, table, rows_per_w, chunk_rows, row0, out_rows):
    _, d = table.shape
    n_chunks = rows_per_w // chunk_rows
    n_outer = n_chunks // _NBUF

    mesh = plsc.VectorSubcoreMesh(core_axis_name="c", subcore_axis_name="s")

    @functools.partial(
        pl.kernel,
        mesh=mesh,
        out_type=jax.ShapeDtypeStruct((out_rows, _LANE, d), jnp.float32),
        scratch_types=(
            [pltpu.VMEM((chunk_rows, _LANE), jnp.int32) for _ in range(_NBUF)]
            + [pltpu.VMEM((chunk_rows, _LANE, d), jnp.float32) for _ in range(_NBUF)]
            + [pltpu.SemaphoreType.DMA for _ in range(3 * _NBUF)]
        ),
        compiler_params=pltpu.CompilerParams(use_tc_tiling_on_sc=False),
    )
    def k(idx_hbm, table_hbm, out_hbm, *scratch):
        idx_v = scratch[:_NBUF]
        rows_v = scratch[_NBUF : 2 * _NBUF]
        sem_idx = scratch[2 * _NBUF : 3 * _NBUF]
        sem_g = scratch[3 * _NBUF : 4 * _NBUF]
        sem_out = scratch[4 * _NBUF : 5 * _NBUF]

        wid = lax.axis_index("s") * 2 + lax.axis_index("c")
        base = wid * rows_per_w

        def idx_copy(c, b):
            return pltpu.make_async_copy(
                idx_hbm.at[pl.ds(row0 + base + c * chunk_rows, chunk_rows)],
                idx_v[b],
                sem_idx[b],
            )

        def out_copy(c, b):
            return pltpu.make_async_copy(
                rows_v[b],
                out_hbm.at[pl.ds(base + c * chunk_rows, chunk_rows)],
                sem_out[b],
            )

        for b in range(_NBUF):
            idx_copy(b, b).start()

        def body(i, carry):
            c0 = i * _NBUF
            handles = []
            for b in range(_NBUF):
                c = c0 + b
                @pl.when(i > 0)
                def _():
                    out_copy(c, b).wait()

                idx_copy(c, b).wait()
                handles.append(
                    [
                        pltpu.async_copy(
                            table_hbm.at[idx_v[b].at[j]],
                            rows_v[b].at[j],
                            sem_g[b],
                        )
                        for j in range(chunk_rows)
                    ]
                )
            for b in range(_NBUF):
                c = c0 + b
                for h in handles[b]:
                    h.wait()
                out_copy(c, b).start()

                @pl.when(c + _NBUF < n_chunks)
                def _():
                    idx_copy(c + _NBUF, b).start()

            return carry

        lax.fori_loop(0, n_outer, body, 0)
        for b in range(_NBUF):
            out_copy(n_chunks - _NBUF + b, b).wait()

    return k(idx2d, table)


def kernel(x, table):
    b, l = x.shape
    v, d = table.shape
    n = b * l
    n_rows = n // _LANE
    rows_per_w = n_rows // _NW
    chunk_rows = 10

    xi = x.astype(jnp.int32)
    rho = ((xi >> 9) << 9) + ((xi & 127) << 2) + ((xi >> 7) & 3)
    idx2d = rho.reshape(n_rows, _LANE)
    t128 = _table_repack(table.T, v, d)
    t_lin = t128.reshape(t128.shape[0] * _LANE // d, d)
    half_rows = n_rows // 2
    rows_per_w_h = half_rows // _NW
    out_h = [
        _sc_gather(idx2d, t_lin, rows_per_w_h, chunk_rows, h * half_rows, half_rows)
        for h in range(2)
    ]
    o2d = [o.reshape(half_rows * d, _LANE) for o in out_h]
    y = _out_repack_half(o2d[0], None, b, l * d, 0)
    y = _out_repack_half(o2d[1], y, b, l * d, 1)
    return y.reshape(l, d, b).transpose(2, 0, 1)


# speedup vs baseline: 1.0551x; 1.0551x over previous
import functools

import jax
import jax.numpy as jnp
from jax import lax
from jax.experimental import pallas as pl
from jax.experimental.pallas import tpu as pltpu
from jax.experimental.pallas import tpu_sc as plsc

_LANE = 128
_NBUF = 2
_NW = 32


def _table_repack_body(t_ref, out_ref):
    x = t_ref[...]
    nb = t_ref.shape[1] // (4 * _LANE)
    for p in range(nb):
        s = jnp.concatenate(
            [
                x[:, 4 * _LANE * p + _LANE * j : 4 * _LANE * p + _LANE * (j + 1)]
                for j in range(4)
            ],
            axis=0,
        )
        out_ref[pl.ds(p * _LANE, _LANE), :] = s.T


def _table_repack(table_t, v, d):
    vb = 32768
    n_steps = -(-v // vb)
    v_pad = n_steps * vb
    return pl.pallas_call(
        _table_repack_body,
        grid=(n_steps,),
        in_specs=[pl.BlockSpec((d, vb), lambda i: (0, i))],
        out_specs=pl.BlockSpec((vb * d // _LANE, _LANE), lambda i: (i, 0)),
        out_shape=jax.ShapeDtypeStruct((v_pad * d // _LANE, _LANE), jnp.float32),
    )(table_t)


def _out_repack_body(in_ref, out_ref):
    bw = out_ref.shape[1]
    qn = out_ref.shape[0] // _LANE
    x3 = in_ref[...].reshape(bw, qn, _LANE)
    x3 = jnp.transpose(x3, (1, 0, 2))
    x3 = jnp.transpose(x3, (0, 2, 1))
    out_ref[...] = x3.reshape(out_ref.shape)


def _out_repack_alias_body(in_ref, y_ref, out_ref):
    del y_ref
    _out_repack_body(in_ref, out_ref)


def _out_repack_half(out2d_h, y_prev, b_total, ld, half):
    bw = 256
    b_half = b_total // 2
    n_steps = b_half // bw
    rows_per_blk = bw * ld // _LANE
    in_spec = pl.BlockSpec((rows_per_blk, _LANE), lambda i: (i, 0))
    if half == 0:
        return pl.pallas_call(
            _out_repack_body,
            grid=(n_steps,),
            in_specs=[in_spec],
            out_specs=pl.BlockSpec((ld, bw), lambda i: (0, i)),
            out_shape=jax.ShapeDtypeStruct((ld, b_total), jnp.float32),
        )(out2d_h)
    return pl.pallas_call(
        _out_repack_alias_body,
        grid=(n_steps,),
        in_specs=[in_spec, pl.BlockSpec(memory_space=pl.ANY)],
        out_specs=pl.BlockSpec((ld, bw), lambda i, off=n_steps: (0, i + off)),
        out_shape=jax.ShapeDtypeStruct((ld, b_total), jnp.float32),
        input_output_aliases={1: 0},
    )(out2d_h, y_prev)


def _sc_gather(idx2d, table, rows_per_w, chunk_rows, row0, out_rows):
    _, d = table.shape
    n_chunks = rows_per_w // chunk_rows
    n_outer = n_chunks // _NBUF

    mesh = plsc.VectorSubcoreMesh(core_axis_name="c", subcore_axis_name="s")

    @functools.partial(
        pl.kernel,
        mesh=mesh,
        out_type=jax.ShapeDtypeStruct((out_rows, _LANE, d), jnp.float32),
        scratch_types=(
            [pltpu.VMEM((chunk_rows, _LANE), jnp.int32) for _ in range(_NBUF)]
            + [pltpu.VMEM((chunk_rows, _LANE, d), jnp.float32) for _ in range(_NBUF)]
            + [pltpu.SemaphoreType.DMA for _ in range(3 * _NBUF)]
        ),
        compiler_params=pltpu.CompilerParams(use_tc_tiling_on_sc=False),
    )
    def k(idx_hbm, table_hbm, out_hbm, *scratch):
        idx_v = scratch[:_NBUF]
        rows_v = scratch[_NBUF : 2 * _NBUF]
        sem_idx = scratch[2 * _NBUF : 3 * _NBUF]
        sem_g = scratch[3 * _NBUF : 4 * _NBUF]
        sem_out = scratch[4 * _NBUF : 5 * _NBUF]

        wid = lax.axis_index("s") * 2 + lax.axis_index("c")
        base = wid * rows_per_w

        def idx_copy(c, b):
            return pltpu.make_async_copy(
                idx_hbm.at[pl.ds(row0 + base + c * chunk_rows, chunk_rows)],
                idx_v[b],
                sem_idx[b],
            )

        def out_copy(c, b):
            return pltpu.make_async_copy(
                rows_v[b],
                out_hbm.at[pl.ds(base + c * chunk_rows, chunk_rows)],
                sem_out[b],
            )

        for b in range(_NBUF):
            idx_copy(b, b).start()

        def body(i, carry):
            c0 = i * _NBUF
            handles = []
            for b in range(_NBUF):
                c = c0 + b
                @pl.when(i > 0)
                def _():
                    out_copy(c, b).wait()

                idx_copy(c, b).wait()
                handles.append(
                    [
                        pltpu.async_copy(
                            table_hbm.at[idx_v[b].at[j]],
                            rows_v[b].at[j],
                            sem_g[b],
                        )
                        for j in range(chunk_rows)
                    ]
                )
            for b in range(_NBUF):
                c = c0 + b
                for h in handles[b]:
                    h.wait()
                out_copy(c, b).start()

                @pl.when(c + _NBUF < n_chunks)
                def _():
                    idx_copy(c + _NBUF, b).start()

            return carry

        lax.fori_loop(0, n_outer, body, 0)
        for b in range(_NBUF):
            out_copy(n_chunks - _NBUF + b, b).wait()

    return k(idx2d, table)


def kernel(x, table):
    b, l = x.shape
    v, d = table.shape
    n = b * l
    n_rows = n // _LANE
    rows_per_w = n_rows // _NW
    chunk_rows = 10

    xi = x.astype(jnp.int32)
    rho = ((xi >> 9) << 9) + ((xi & 127) << 2) + ((xi >> 7) & 3)
    idx2d = rho.reshape(n_rows, _LANE)
    t128 = _table_repack(table.T, v, d)
    t_lin = t128.reshape(t128.shape[0] * _LANE // d, d)
    half_rows = n_rows // 2
    rows_per_w_h = half_rows // _NW
    out_h = [
        _sc_gather(idx2d, t_lin, rows_per_w_h, chunk_rows, h * half_rows, half_rows)
        for h in range(2)
    ]
    o2d = [o.reshape(half_rows * d, _LANE) for o in out_h]
    y = _out_repack_half(o2d[0], None, b, l * d, 0)
    y = _out_repack_half(o2d[1], y, b, l * d, 1)
    return y.reshape(l, d, b).transpose(2, 0, 1)


# speedup vs baseline: 1.0592x; 1.0039x over previous
import functools

import jax
import jax.numpy as jnp
from jax import lax
from jax.experimental import pallas as pl
from jax.experimental.pallas import tpu as pltpu
from jax.experimental.pallas import tpu_sc as plsc

_LANE = 128
_NBUF = 2
_NW = 32


def _table_repack_body(t_ref, out_ref):
    x = t_ref[...]
    nb = t_ref.shape[1] // (4 * _LANE)
    for p in range(nb):
        s = jnp.concatenate(
            [
                x[:, 4 * _LANE * p + _LANE * j : 4 * _LANE * p + _LANE * (j + 1)]
                for j in range(4)
            ],
            axis=0,
        )
        out_ref[pl.ds(p * _LANE, _LANE), :] = s.T


def _table_repack(table_t, v, d):
    vb = 65536
    n_steps = -(-v // vb)
    v_pad = n_steps * vb
    return pl.pallas_call(
        _table_repack_body,
        grid=(n_steps,),
        in_specs=[pl.BlockSpec((d, vb), lambda i: (0, i))],
        out_specs=pl.BlockSpec((vb * d // _LANE, _LANE), lambda i: (i, 0)),
        out_shape=jax.ShapeDtypeStruct((v_pad * d // _LANE, _LANE), jnp.float32),
    )(table_t)


def _out_repack_body(in_ref, out_ref):
    bw = out_ref.shape[1]
    qn = out_ref.shape[0] // _LANE
    x3 = in_ref[...].reshape(bw, qn, _LANE)
    x3 = jnp.transpose(x3, (1, 0, 2))
    x3 = jnp.transpose(x3, (0, 2, 1))
    out_ref[...] = x3.reshape(out_ref.shape)


def _out_repack_alias_body(in_ref, y_ref, out_ref):
    del y_ref
    _out_repack_body(in_ref, out_ref)


def _out_repack_half(out2d_h, y_prev, b_total, ld, half):
    bw = 256
    b_half = b_total // 2
    n_steps = b_half // bw
    rows_per_blk = bw * ld // _LANE
    in_spec = pl.BlockSpec((rows_per_blk, _LANE), lambda i: (i, 0))
    if half == 0:
        return pl.pallas_call(
            _out_repack_body,
            grid=(n_steps,),
            in_specs=[in_spec],
            out_specs=pl.BlockSpec((ld, bw), lambda i: (0, i)),
            out_shape=jax.ShapeDtypeStruct((ld, b_total), jnp.float32),
        )(out2d_h)
    return pl.pallas_call(
        _out_repack_alias_body,
        grid=(n_steps,),
        in_specs=[in_spec, pl.BlockSpec(memory_space=pl.ANY)],
        out_specs=pl.BlockSpec((ld, bw), lambda i, off=n_steps: (0, i + off)),
        out_shape=jax.ShapeDtypeStruct((ld, b_total), jnp.float32),
        input_output_aliases={1: 0},
    )(out2d_h, y_prev)


def _sc_gather(idx2d, table, rows_per_w, chunk_rows, row0, out_rows):
    _, d = table.shape
    n_chunks = rows_per_w // chunk_rows
    n_outer = n_chunks // _NBUF

    mesh = plsc.VectorSubcoreMesh(core_axis_name="c", subcore_axis_name="s")

    @functools.partial(
        pl.kernel,
        mesh=mesh,
        out_type=jax.ShapeDtypeStruct((out_rows, _LANE, d), jnp.float32),
        scratch_types=(
            [pltpu.VMEM((chunk_rows, _LANE), jnp.int32) for _ in range(_NBUF)]
            + [pltpu.VMEM((chunk_rows, _LANE, d), jnp.float32) for _ in range(_NBUF)]
            + [pltpu.SemaphoreType.DMA for _ in range(3 * _NBUF)]
        ),
        compiler_params=pltpu.CompilerParams(use_tc_tiling_on_sc=False),
    )
    def k(idx_hbm, table_hbm, out_hbm, *scratch):
        idx_v = scratch[:_NBUF]
        rows_v = scratch[_NBUF : 2 * _NBUF]
        sem_idx = scratch[2 * _NBUF : 3 * _NBUF]
        sem_g = scratch[3 * _NBUF : 4 * _NBUF]
        sem_out = scratch[4 * _NBUF : 5 * _NBUF]

        wid = lax.axis_index("s") * 2 + lax.axis_index("c")
        base = wid * rows_per_w

        def idx_copy(c, b):
            return pltpu.make_async_copy(
                idx_hbm.at[pl.ds(row0 + base + c * chunk_rows, chunk_rows)],
                idx_v[b],
                sem_idx[b],
            )

        def out_copy(c, b):
            return pltpu.make_async_copy(
                rows_v[b],
                out_hbm.at[pl.ds(base + c * chunk_rows, chunk_rows)],
                sem_out[b],
            )

        for b in range(_NBUF):
            idx_copy(b, b).start()

        def body(i, carry):
            c0 = i * _NBUF
            handles = []
            for b in range(_NBUF):
                c = c0 + b
                @pl.when(i > 0)
                def _():
                    out_copy(c, b).wait()

                idx_copy(c, b).wait()
                handles.append(
                    [
                        pltpu.async_copy(
                            table_hbm.at[idx_v[b].at[j]],
                            rows_v[b].at[j],
                            sem_g[b],
                        )
                        for j in range(chunk_rows)
                    ]
                )
            for b in range(_NBUF):
                c = c0 + b
                for h in handles[b]:
                    h.wait()
                out_copy(c, b).start()

                @pl.when(c + _NBUF < n_chunks)
                def _():
                    idx_copy(c + _NBUF, b).start()

            return carry

        lax.fori_loop(0, n_outer, body, 0)
        for b in range(_NBUF):
            out_copy(n_chunks - _NBUF + b, b).wait()

    return k(idx2d, table)


def kernel(x, table):
    b, l = x.shape
    v, d = table.shape
    n = b * l
    n_rows = n // _LANE
    rows_per_w = n_rows // _NW
    chunk_rows = 10

    xi = x.astype(jnp.int32)
    rho = ((xi >> 9) << 9) + ((xi & 127) << 2) + ((xi >> 7) & 3)
    idx2d = rho.reshape(n_rows, _LANE)
    t128 = _table_repack(table.T, v, d)
    t_lin = t128.reshape(t128.shape[0] * _LANE // d, d)
    half_rows = n_rows // 2
    rows_per_w_h = half_rows // _NW
    out_h = [
        _sc_gather(idx2d, t_lin, rows_per_w_h, chunk_rows, h * half_rows, half_rows)
        for h in range(2)
    ]
    o2d = [o.reshape(half_rows * d, _LANE) for o in out_h]
    y = _out_repack_half(o2d[0], None, b, l * d, 0)
    y = _out_repack_half(o2d[1], y, b, l * d, 1)
    return y.reshape(l, d, b).transpose(2, 0, 1)


# speedup vs baseline: 1.0630x; 1.0036x over previous
import functools

import jax
import jax.numpy as jnp
from jax import lax
from jax.experimental import pallas as pl
from jax.experimental.pallas import tpu as pltpu
from jax.experimental.pallas import tpu_sc as plsc

_LANE = 128
_NBUF = 2
_NW = 32


def _table_repack_body(t_ref, out_ref):
    x = t_ref[...]
    nb = t_ref.shape[1] // (4 * _LANE)
    for p in range(nb):
        s = jnp.concatenate(
            [
                x[:, 4 * _LANE * p + _LANE * j : 4 * _LANE * p + _LANE * (j + 1)]
                for j in range(4)
            ],
            axis=0,
        )
        out_ref[pl.ds(p * _LANE, _LANE), :] = s.T


def _table_repack(table_t, v, d):
    vb = 65536
    n_steps = -(-v // vb)
    v_pad = n_steps * vb
    return pl.pallas_call(
        _table_repack_body,
        grid=(n_steps,),
        in_specs=[pl.BlockSpec((d, vb), lambda i: (0, i))],
        out_specs=pl.BlockSpec((vb * d // _LANE, _LANE), lambda i: (i, 0)),
        out_shape=jax.ShapeDtypeStruct((v_pad * d // _LANE, _LANE), jnp.float32),
    )(table_t)


def _out_repack_body(in_ref, out_ref):
    bw = out_ref.shape[1]
    qn = out_ref.shape[0] // _LANE
    x3 = in_ref[...].reshape(bw, qn, _LANE)
    x3 = jnp.transpose(x3, (1, 0, 2))
    x3 = jnp.transpose(x3, (0, 2, 1))
    out_ref[...] = x3.reshape(out_ref.shape)


def _out_repack_alias_body(in_ref, y_ref, out_ref):
    del y_ref
    _out_repack_body(in_ref, out_ref)


def _out_repack_part(out2d_h, y_prev, b_total, ld, part, nparts):
    bw = 256
    b_part = b_total // nparts
    n_steps = b_part // bw
    rows_per_blk = bw * ld // _LANE
    in_spec = pl.BlockSpec((rows_per_blk, _LANE), lambda i: (i, 0))
    if part == 0:
        return pl.pallas_call(
            _out_repack_body,
            grid=(n_steps,),
            in_specs=[in_spec],
            out_specs=pl.BlockSpec((ld, bw), lambda i: (0, i)),
            out_shape=jax.ShapeDtypeStruct((ld, b_total), jnp.float32),
        )(out2d_h)
    return pl.pallas_call(
        _out_repack_alias_body,
        grid=(n_steps,),
        in_specs=[in_spec, pl.BlockSpec(memory_space=pl.ANY)],
        out_specs=pl.BlockSpec(
            (ld, bw), lambda i, off=part * n_steps: (0, i + off)
        ),
        out_shape=jax.ShapeDtypeStruct((ld, b_total), jnp.float32),
        input_output_aliases={1: 0},
    )(out2d_h, y_prev)


def _sc_gather(idx2d, table, rows_per_w, chunk_rows, row0, out_rows):
    _, d = table.shape
    n_chunks = rows_per_w // chunk_rows
    n_outer = n_chunks // _NBUF

    mesh = plsc.VectorSubcoreMesh(core_axis_name="c", subcore_axis_name="s")

    @functools.partial(
        pl.kernel,
        mesh=mesh,
        out_type=jax.ShapeDtypeStruct((out_rows, _LANE, d), jnp.float32),
        scratch_types=(
            [pltpu.VMEM((chunk_rows, _LANE), jnp.int32) for _ in range(_NBUF)]
            + [pltpu.VMEM((chunk_rows, _LANE, d), jnp.float32) for _ in range(_NBUF)]
            + [pltpu.SemaphoreType.DMA for _ in range(3 * _NBUF)]
        ),
        compiler_params=pltpu.CompilerParams(use_tc_tiling_on_sc=False),
    )
    def k(idx_hbm, table_hbm, out_hbm, *scratch):
        idx_v = scratch[:_NBUF]
        rows_v = scratch[_NBUF : 2 * _NBUF]
        sem_idx = scratch[2 * _NBUF : 3 * _NBUF]
        sem_g = scratch[3 * _NBUF : 4 * _NBUF]
        sem_out = scratch[4 * _NBUF : 5 * _NBUF]

        wid = lax.axis_index("s") * 2 + lax.axis_index("c")
        base = wid * rows_per_w

        def idx_copy(c, b):
            return pltpu.make_async_copy(
                idx_hbm.at[pl.ds(row0 + base + c * chunk_rows, chunk_rows)],
                idx_v[b],
                sem_idx[b],
            )

        def out_copy(c, b):
            return pltpu.make_async_copy(
                rows_v[b],
                out_hbm.at[pl.ds(base + c * chunk_rows, chunk_rows)],
                sem_out[b],
            )

        for b in range(_NBUF):
            idx_copy(b, b).start()

        def body(i, carry):
            c0 = i * _NBUF
            handles = []
            for b in range(_NBUF):
                c = c0 + b
                @pl.when(i > 0)
                def _():
                    out_copy(c, b).wait()

                idx_copy(c, b).wait()
                handles.append(
                    [
                        pltpu.async_copy(
                            table_hbm.at[idx_v[b].at[j]],
                            rows_v[b].at[j],
                            sem_g[b],
                        )
                        for j in range(chunk_rows)
                    ]
                )
            for b in range(_NBUF):
                c = c0 + b
                for h in handles[b]:
                    h.wait()
                out_copy(c, b).start()

                @pl.when(c + _NBUF < n_chunks)
                def _():
                    idx_copy(c + _NBUF, b).start()

            return carry

        lax.fori_loop(0, n_outer, body, 0)
        for b in range(_NBUF):
            out_copy(n_chunks - _NBUF + b, b).wait()

    return k(idx2d, table)


def kernel(x, table):
    b, l = x.shape
    v, d = table.shape
    n = b * l
    n_rows = n // _LANE
    rows_per_w = n_rows // _NW
    chunk_rows = 5

    xi = x.astype(jnp.int32)
    rho = ((xi >> 9) << 9) + ((xi & 127) << 2) + ((xi >> 7) & 3)
    idx2d = rho.reshape(n_rows, _LANE)
    t128 = _table_repack(table.T, v, d)
    t_lin = t128.reshape(t128.shape[0] * _LANE // d, d)
    nparts = 4
    part_rows = n_rows // nparts
    rows_per_w_p = part_rows // _NW
    out_p = [
        _sc_gather(idx2d, t_lin, rows_per_w_p, chunk_rows, h * part_rows, part_rows)
        for h in range(nparts)
    ]
    y = None
    for h in range(nparts):
        o2d = out_p[h].reshape(part_rows * d, _LANE)
        y = _out_repack_part(o2d, y, b, l * d, h, nparts)
    return y.reshape(l, d, b).transpose(2, 0, 1)


# speedup vs baseline: 1.0706x; 1.0071x over previous
import functools

import jax
import jax.numpy as jnp
from jax import lax
from jax.experimental import pallas as pl
from jax.experimental.pallas import tpu as pltpu
from jax.experimental.pallas import tpu_sc as plsc

_LANE = 128
_NBUF = 5
_NW = 32


def _table_repack_body(t_ref, out_ref):
    x = t_ref[...]
    nb = t_ref.shape[1] // (4 * _LANE)
    for p in range(nb):
        s = jnp.concatenate(
            [
                x[:, 4 * _LANE * p + _LANE * j : 4 * _LANE * p + _LANE * (j + 1)]
                for j in range(4)
            ],
            axis=0,
        )
        out_ref[pl.ds(p * _LANE, _LANE), :] = s.T


def _table_repack(table_t, v, d):
    vb = 65536
    n_steps = -(-v // vb)
    v_pad = n_steps * vb
    return pl.pallas_call(
        _table_repack_body,
        grid=(n_steps,),
        in_specs=[pl.BlockSpec((d, vb), lambda i: (0, i))],
        out_specs=pl.BlockSpec((vb * d // _LANE, _LANE), lambda i: (i, 0)),
        out_shape=jax.ShapeDtypeStruct((v_pad * d // _LANE, _LANE), jnp.float32),
    )(table_t)


def _out_repack_body(in_ref, out_ref):
    bw = out_ref.shape[1]
    qn = out_ref.shape[0] // _LANE
    x3 = in_ref[...].reshape(bw, qn, _LANE)
    x3 = jnp.transpose(x3, (1, 0, 2))
    x3 = jnp.transpose(x3, (0, 2, 1))
    out_ref[...] = x3.reshape(out_ref.shape)


def _out_repack_alias_body(in_ref, y_ref, out_ref):
    del y_ref
    _out_repack_body(in_ref, out_ref)


def _out_repack_part(out2d_h, y_prev, b_total, ld, part, nparts):
    bw = 256
    b_part = b_total // nparts
    n_steps = b_part // bw
    rows_per_blk = bw * ld // _LANE
    in_spec = pl.BlockSpec((rows_per_blk, _LANE), lambda i: (i, 0))
    if part == 0:
        return pl.pallas_call(
            _out_repack_body,
            grid=(n_steps,),
            in_specs=[in_spec],
            out_specs=pl.BlockSpec((ld, bw), lambda i: (0, i)),
            out_shape=jax.ShapeDtypeStruct((ld, b_total), jnp.float32),
        )(out2d_h)
    return pl.pallas_call(
        _out_repack_alias_body,
        grid=(n_steps,),
        in_specs=[in_spec, pl.BlockSpec(memory_space=pl.ANY)],
        out_specs=pl.BlockSpec(
            (ld, bw), lambda i, off=part * n_steps: (0, i + off)
        ),
        out_shape=jax.ShapeDtypeStruct((ld, b_total), jnp.float32),
        input_output_aliases={1: 0},
    )(out2d_h, y_prev)


def _sc_gather(idx2d, table, rows_per_w, chunk_rows, row0, out_rows):
    _, d = table.shape
    n_chunks = rows_per_w // chunk_rows
    n_outer = n_chunks // _NBUF

    mesh = plsc.VectorSubcoreMesh(core_axis_name="c", subcore_axis_name="s")

    @functools.partial(
        pl.kernel,
        mesh=mesh,
        out_type=jax.ShapeDtypeStruct((out_rows, _LANE, d), jnp.float32),
        scratch_types=(
            [pltpu.VMEM((chunk_rows, _LANE), jnp.int32) for _ in range(_NBUF)]
            + [pltpu.VMEM((chunk_rows, _LANE, d), jnp.float32) for _ in range(_NBUF)]
            + [pltpu.SemaphoreType.DMA for _ in range(3 * _NBUF)]
        ),
        compiler_params=pltpu.CompilerParams(use_tc_tiling_on_sc=False),
    )
    def k(idx_hbm, table_hbm, out_hbm, *scratch):
        idx_v = scratch[:_NBUF]
        rows_v = scratch[_NBUF : 2 * _NBUF]
        sem_idx = scratch[2 * _NBUF : 3 * _NBUF]
        sem_g = scratch[3 * _NBUF : 4 * _NBUF]
        sem_out = scratch[4 * _NBUF : 5 * _NBUF]

        wid = lax.axis_index("s") * 2 + lax.axis_index("c")
        base = wid * rows_per_w

        def idx_copy(c, b):
            return pltpu.make_async_copy(
                idx_hbm.at[pl.ds(row0 + base + c * chunk_rows, chunk_rows)],
                idx_v[b],
                sem_idx[b],
            )

        def out_copy(c, b):
            return pltpu.make_async_copy(
                rows_v[b],
                out_hbm.at[pl.ds(base + c * chunk_rows, chunk_rows)],
                sem_out[b],
            )

        for b in range(_NBUF):
            idx_copy(b, b).start()

        def body(i, carry):
            c0 = i * _NBUF
            handles = []
            for b in range(_NBUF):
                c = c0 + b
                @pl.when(i > 0)
                def _():
                    out_copy(c, b).wait()

                idx_copy(c, b).wait()
                handles.append(
                    [
                        pltpu.async_copy(
                            table_hbm.at[idx_v[b].at[j]],
                            rows_v[b].at[j],
                            sem_g[b],
                        )
                        for j in range(chunk_rows)
                    ]
                )
            for b in range(_NBUF):
                c = c0 + b
                for h in handles[b]:
                    h.wait()
                out_copy(c, b).start()

                @pl.when(c + _NBUF < n_chunks)
                def _():
                    idx_copy(c + _NBUF, b).start()

            return carry

        lax.fori_loop(0, n_outer, body, 0)
        for b in range(_NBUF):
            out_copy(n_chunks - _NBUF + b, b).wait()

    return k(idx2d, table)


def kernel(x, table):
    b, l = x.shape
    v, d = table.shape
    n = b * l
    n_rows = n // _LANE
    rows_per_w = n_rows // _NW
    chunk_rows = 5

    xi = x.astype(jnp.int32)
    rho = ((xi >> 9) << 9) + ((xi & 127) << 2) + ((xi >> 7) & 3)
    idx2d = rho.reshape(n_rows, _LANE)
    t128 = _table_repack(table.T, v, d)
    t_lin = t128.reshape(t128.shape[0] * _LANE // d, d)
    nparts = 4
    part_rows = n_rows // nparts
    rows_per_w_p = part_rows // _NW
    out_p = [
        _sc_gather(idx2d, t_lin, rows_per_w_p, chunk_rows, h * part_rows, part_rows)
        for h in range(nparts)
    ]
    y = None
    for h in range(nparts):
        o2d = out_p[h].reshape(part_rows * d, _LANE)
        y = _out_repack_part(o2d, y, b, l * d, h, nparts)
    return y.reshape(l, d, b).transpose(2, 0, 1)
